# baseline (speedup 1.0000x reference)
import functools

import jax
import jax.numpy as jnp
from jax import lax
from jax.experimental import pallas as pl
from jax.experimental.pallas import tpu as pltpu
from jax.experimental.pallas import tpu_sc as plsc

_EPS = 1e-8
_BIG = 3.4e38


def _feat_body(x1_ref, w1_ref, x2_ref, w2_ref,
               z1_ref, z2_ref, s1_ref, q1_ref, s2_ref, q2_ref):
    b = pl.program_id(0)
    z1 = jax.lax.dot_general(x1_ref[0], w1_ref[...], (((0,), (1,)), ((), ())),
                             preferred_element_type=jnp.float32)
    z1_ref[0] = z1
    z2 = jax.lax.dot_general(w2_ref[...], x2_ref[0], (((1,), (0,)), ((), ())),
                             preferred_element_type=jnp.float32)
    z2_ref[0] = z2
    s1 = jnp.sum(z1, axis=0, keepdims=True)
    q1 = jnp.sum(z1 * z1, axis=0, keepdims=True)
    s2 = jnp.sum(z2, axis=1, keepdims=True)
    q2 = jnp.sum(z2 * z2, axis=1, keepdims=True)

    @pl.when(b == 0)
    def _init():
        s1_ref[...] = s1
        q1_ref[...] = q1
        s2_ref[...] = s2
        q2_ref[...] = q2

    @pl.when(b != 0)
    def _acc():
        s1_ref[...] += s1
        q1_ref[...] += q1
        s2_ref[...] += s2
        q2_ref[...] += q2


def _knn_idx_body(n1, tn2, nt2, p1_ref, p2_ref, z1_ref, ab1_ref,
                  idx_ref, wexp_ref, f1n_ref, f1n_scr, p1sq_ref):
    b = pl.program_id(0)

    @pl.when(pl.program_id(1) == 0)
    def _prep():
        a1 = ab1_ref[0:1, :]
        b1 = ab1_ref[1:2, :]
        f1n_scr[...] = jnp.maximum(z1_ref[0] * a1 + b1, 0.0)
        f1n_ref[0] = f1n_scr[...]
        p1 = p1_ref[0]
        ones = jnp.ones((1, 3), jnp.float32)
        p1sq_ref[...] = jax.lax.dot_general(
            ones, p1 * p1, (((1,), (1,)), ((), ())),
            precision=jax.lax.Precision.HIGHEST,
            preferred_element_type=jnp.float32)

    p2t = p2_ref[0]
    p2sq = jnp.sum(p2t * p2t, axis=1, keepdims=True)
    dd = jax.lax.dot_general(p2t, p1_ref[0], (((1,), (1,)), ((), ())),
                             preferred_element_type=jnp.float32)
    d = p2sq + p1sq_ref[...] - 2.0 * dd

    big = jnp.full((tn2, 128), _BIG, jnp.float32)
    r1, r2, r3 = big, big, big
    for c in range(n1 // 128):
        x = d[:, c * 128:(c + 1) * 128]
        hi1 = jnp.maximum(r1, x)
        r1 = jnp.minimum(r1, x)
        hi2 = jnp.maximum(r2, hi1)
        r2 = jnp.minimum(r2, hi1)
        r3 = jnp.minimum(r3, hi2)
    r = jnp.concatenate([r1, r2, r3], axis=1)
    io = jax.lax.broadcasted_iota(jnp.int32, (tn2, 384), 1)
    ms = []
    for _ in range(3):
        m = jnp.min(r, axis=1, keepdims=True)
        i = jnp.min(jnp.where(r == m, io, 384), axis=1, keepdims=True)
        ms.append(m)
        r = jnp.where(io == i, _BIG, r)

    lane = jax.lax.broadcasted_iota(jnp.int32, (tn2, n1), 1)
    i1 = jnp.min(jnp.where(d == ms[0], lane, n1), axis=1, keepdims=True)
    thr2 = jnp.where(ms[1] == ms[0], i1, -1)
    i2 = jnp.min(jnp.where((d == ms[1]) & (lane > thr2), lane, n1),
                 axis=1, keepdims=True)
    thr3 = jnp.where(ms[2] == ms[1], i2, -1)
    i3 = jnp.min(jnp.where((d == ms[2]) & (lane > thr3), lane, n1),
                 axis=1, keepdims=True)

    base = b * n1
    gidx = jnp.concatenate([i1, i2, i3], axis=1) + base
    idx_ref[0] = gidx.T

    w1 = 1.0 / (jnp.maximum(ms[0], 0.0) + _EPS)
    w2 = 1.0 / (jnp.maximum(ms[1], 0.0) + _EPS)
    w3 = 1.0 / (jnp.maximum(ms[2], 0.0) + _EPS)
    norm = w1 + w2 + w3
    wexp_ref[0] = jnp.concatenate(
        [jnp.broadcast_to(w1 / norm, (tn2, 16)),
         jnp.broadcast_to(w2 / norm, (tn2, 16)),
         jnp.broadcast_to(w3 / norm, (tn2, 16))], axis=1)


def _skip_add_body(z2_ref, ab2_ref, up_ref, y_ref):
    a2 = ab2_ref[:, 0:1]
    b2 = ab2_ref[:, 1:2]
    f2 = jnp.maximum(z2_ref[0] * a2 + b2, 0.0)
    y_ref[0] = f2 + up_ref[0].T


def _sc_interp(table, idx, wexp, qtot, tn2, cout):
    nblk = idx.shape[0]
    info = plsc.get_sparse_core_info()
    nw = info.num_cores * info.num_subcores
    blk_per_w = nblk // nw
    nh = tn2 // 128
    mesh = plsc.VectorSubcoreMesh(core_axis_name="c", subcore_axis_name="s")
    iw = 3 * tn2
    ww = tn2 * 48
    ow = tn2 * cout

    @functools.partial(
        pl.kernel, mesh=mesh,
        out_type=jax.ShapeDtypeStruct((qtot * cout,), jnp.float32),
        scratch_types=[
            pltpu.VMEM((iw,), jnp.int32),
            pltpu.VMEM((ww,), jnp.float32),
            pltpu.VMEM((128, cout), jnp.float32),
            pltpu.VMEM((ow,), jnp.float32),
            pltpu.SemaphoreType.DMA,
        ],
    )
    def k(table_hbm, idx_hbm, wexp_hbm, out_hbm, idx_v, w_v, rows_v, acc_v, sem):
        wid = lax.axis_index("s") * info.num_cores + lax.axis_index("c")
        ci = lax.iota(jnp.int32, 16)
        for blk in range(blk_per_w):
            bt = wid * blk_per_w + blk
            pltpu.sync_copy(idx_hbm.at[pl.ds(bt * iw, iw)], idx_v)
            pltpu.sync_copy(wexp_hbm.at[pl.ds(bt * ww, ww)], w_v)
            for kk in range(3):
                for h in range(nh):
                    pltpu.async_copy(
                        table_hbm.at[idx_v.at[pl.ds(kk * tn2 + h * 128, 128)]],
                        rows_v, sem).wait()

                    def qbody(q, _):
                        wv = w_v[pl.ds((h * 128 + q) * 48 + kk * 16, 16)]
                        for g in range(cout // 16):
                            seg = rows_v[q, pl.ds(g * 16, 16)] * wv
                            o = (h * 128 + q) * cout + g * 16
                            if kk == 0:
                                acc_v[pl.ds(o, 16)] = seg
                            else:
                                acc_v[pl.ds(o, 16)] += seg
                        return 0

                    lax.fori_loop(0, 128, qbody, 0)
            pltpu.sync_copy(acc_v, out_hbm.at[pl.ds(bt * ow, ow)])

    return k(table, idx.reshape(-1), wexp.reshape(-1))


def kernel(p1, x1, p2, x2, W1, gamma1, beta1, W2, gamma2, beta2):
    B, N1, _ = p1.shape
    _, Cin, _ = x1.shape
    _, Cskip, N2 = x2.shape
    Cout = W1.shape[0]
    TN2 = 256
    nt2 = N2 // TN2

    z1, z2, s1, q1, s2, q2 = pl.pallas_call(
        _feat_body,
        grid=(B,),
        in_specs=[
            pl.BlockSpec((1, Cin, N1), lambda b: (b, 0, 0)),
            pl.BlockSpec((Cout, Cin), lambda b: (0, 0)),
            pl.BlockSpec((1, Cskip, N2), lambda b: (b, 0, 0)),
            pl.BlockSpec((Cout, Cskip), lambda b: (0, 0)),
        ],
        out_specs=[
            pl.BlockSpec((1, N1, Cout), lambda b: (b, 0, 0)),
            pl.BlockSpec((1, Cout, N2), lambda b: (b, 0, 0)),
            pl.BlockSpec((1, Cout), lambda b: (0, 0)),
            pl.BlockSpec((1, Cout), lambda b: (0, 0)),
            pl.BlockSpec((Cout, 1), lambda b: (0, 0)),
            pl.BlockSpec((Cout, 1), lambda b: (0, 0)),
        ],
        out_shape=[
            jax.ShapeDtypeStruct((B, N1, Cout), jnp.float32),
            jax.ShapeDtypeStruct((B, Cout, N2), jnp.float32),
            jax.ShapeDtypeStruct((1, Cout), jnp.float32),
            jax.ShapeDtypeStruct((1, Cout), jnp.float32),
            jax.ShapeDtypeStruct((Cout, 1), jnp.float32),
            jax.ShapeDtypeStruct((Cout, 1), jnp.float32),
        ],
    )(x1, W1, x2, W2)

    cnt1 = jnp.float32(B * N1)
    mean1 = s1 / cnt1
    var1 = q1 / cnt1 - mean1 * mean1
    a1 = gamma1[None, :] / jnp.sqrt(var1 + 1e-5)
    b1 = beta1[None, :] - mean1 * a1
    ab1 = jnp.concatenate([a1, b1], axis=0)

    cnt2 = jnp.float32(B * N2)
    mean2 = s2 / cnt2
    var2 = q2 / cnt2 - mean2 * mean2
    a2 = gamma2[:, None] / jnp.sqrt(var2 + 1e-5)
    b2 = beta2[:, None] - mean2 * a2
    ab2 = jnp.concatenate([a2, b2], axis=1)

    idx, wexp, f1n = pl.pallas_call(
        functools.partial(_knn_idx_body, N1, TN2, nt2),
        grid=(B, nt2),
        in_specs=[
            pl.BlockSpec((1, N1, 3), lambda b, t: (b, 0, 0)),
            pl.BlockSpec((1, TN2, 3), lambda b, t: (b, t, 0)),
            pl.BlockSpec((1, N1, Cout), lambda b, t: (b, 0, 0)),
            pl.BlockSpec((2, Cout), lambda b, t: (0, 0)),
        ],
        out_specs=[
            pl.BlockSpec((1, 3, TN2), lambda b, t: (b * (N2 // TN2) + t, 0, 0)),
            pl.BlockSpec((1, TN2, 48), lambda b, t: (b * (N2 // TN2) + t, 0, 0)),
            pl.BlockSpec((1, N1, Cout), lambda b, t: (b, 0, 0)),
        ],
        out_shape=[
            jax.ShapeDtypeStruct((B * nt2, 3, TN2), jnp.int32),
            jax.ShapeDtypeStruct((B * nt2, TN2, 48), jnp.float32),
            jax.ShapeDtypeStruct((B, N1, Cout), jnp.float32),
        ],
        scratch_shapes=[pltpu.VMEM((N1, Cout), jnp.float32),
                        pltpu.VMEM((1, N1), jnp.float32)],
    )(p1, p2, z1, ab1)

    up = _sc_interp(f1n.reshape(B * N1, Cout), idx, wexp,
                    B * N2, TN2, Cout)

    y = pl.pallas_call(
        _skip_add_body,
        grid=(B, nt2),
        in_specs=[
            pl.BlockSpec((1, Cout, TN2), lambda b, t: (b, 0, t)),
            pl.BlockSpec((Cout, 2), lambda b, t: (0, 0)),
            pl.BlockSpec((1, TN2, Cout), lambda b, t: (b * (N2 // TN2) + t, 0, 0)),
        ],
        out_specs=pl.BlockSpec((1, Cout, TN2), lambda b, t: (b, 0, t)),
        out_shape=jax.ShapeDtypeStruct((B, Cout, N2), jnp.float32),
    )(z2, ab2, up.reshape(B * nt2, TN2, Cout))

    return (p2, y)


# speedup vs baseline: 1.0370x; 1.0370x over previous
import functools

import jax
import jax.numpy as jnp
from jax import lax
from jax.experimental import pallas as pl
from jax.experimental.pallas import tpu as pltpu
from jax.experimental.pallas import tpu_sc as plsc

_EPS = 1e-8
_BIG = 3.4e38


def _feat_body(x1_ref, w1_ref, x2_ref, w2_ref,
               z1_ref, z2_ref, s1_ref, q1_ref, s2_ref, q2_ref):
    b = pl.program_id(0)
    z1 = jax.lax.dot_general(x1_ref[0], w1_ref[...], (((0,), (1,)), ((), ())),
                             preferred_element_type=jnp.float32)
    z1_ref[0] = z1
    z2 = jax.lax.dot_general(w2_ref[...], x2_ref[0], (((1,), (0,)), ((), ())),
                             preferred_element_type=jnp.float32)
    z2_ref[0] = z2
    s1 = jnp.sum(z1, axis=0, keepdims=True)
    q1 = jnp.sum(z1 * z1, axis=0, keepdims=True)
    s2 = jnp.sum(z2, axis=1, keepdims=True)
    q2 = jnp.sum(z2 * z2, axis=1, keepdims=True)

    @pl.when(b == 0)
    def _init():
        s1_ref[...] = s1
        q1_ref[...] = q1
        s2_ref[...] = s2
        q2_ref[...] = q2

    @pl.when(b != 0)
    def _acc():
        s1_ref[...] += s1
        q1_ref[...] += q1
        s2_ref[...] += s2
        q2_ref[...] += q2


def _knn_idx_body(n1, tn2, nt2, p1_ref, p2_ref, z1_ref, ab1_ref,
                  idx_ref, wexp_ref, f1n_ref, f1n_scr, p1sq_ref):
    b = pl.program_id(0)

    @pl.when(pl.program_id(1) == 0)
    def _prep():
        a1 = ab1_ref[0:1, :]
        b1 = ab1_ref[1:2, :]
        f1n_scr[...] = jnp.maximum(z1_ref[0] * a1 + b1, 0.0)
        f1n_ref[0] = f1n_scr[...]
        p1 = p1_ref[0]
        ones = jnp.ones((1, 3), jnp.float32)
        p1sq_ref[...] = jax.lax.dot_general(
            ones, p1 * p1, (((1,), (1,)), ((), ())),
            precision=jax.lax.Precision.HIGHEST,
            preferred_element_type=jnp.float32)

    p2t = p2_ref[0]
    p2sq = jnp.sum(p2t * p2t, axis=1, keepdims=True)
    dd = jax.lax.dot_general(p2t, p1_ref[0], (((1,), (1,)), ((), ())),
                             preferred_element_type=jnp.float32)
    d = p2sq + p1sq_ref[...] - 2.0 * dd

    big = jnp.full((tn2, 128), _BIG, jnp.float32)
    r1, r2, r3 = big, big, big
    for c in range(n1 // 128):
        x = d[:, c * 128:(c + 1) * 128]
        hi1 = jnp.maximum(r1, x)
        r1 = jnp.minimum(r1, x)
        hi2 = jnp.maximum(r2, hi1)
        r2 = jnp.minimum(r2, hi1)
        r3 = jnp.minimum(r3, hi2)
    r = jnp.concatenate([r1, r2, r3], axis=1)
    io = jax.lax.broadcasted_iota(jnp.int32, (tn2, 384), 1)
    ms = []
    for _ in range(3):
        m = jnp.min(r, axis=1, keepdims=True)
        i = jnp.min(jnp.where(r == m, io, 384), axis=1, keepdims=True)
        ms.append(m)
        r = jnp.where(io == i, _BIG, r)

    lane128 = jax.lax.broadcasted_iota(jnp.int32, (tn2, 128), 1)
    nfull = jnp.full((tn2, 128), n1, jnp.int32)
    j1, j2, j3 = nfull, nfull, nfull
    for c in range(n1 // 128):
        x = d[:, c * 128:(c + 1) * 128]
        lc = lane128 + (c * 128)
        j1 = jnp.minimum(j1, jnp.where(x == ms[0], lc, n1))
        j2 = jnp.minimum(j2, jnp.where(x == ms[1], lc, n1))
        j3 = jnp.minimum(j3, jnp.where(x == ms[2], lc, n1))
    i1 = jnp.min(j1, axis=1, keepdims=True)
    i2 = jnp.min(j2, axis=1, keepdims=True)
    i3 = jnp.min(j3, axis=1, keepdims=True)

    base = b * n1
    gidx = jnp.concatenate([i1, i2, i3], axis=1) + base
    idx_ref[0] = gidx.T

    dup = jnp.any((ms[1] == ms[0]) | (ms[2] == ms[1]))

    @pl.when(dup)
    def _fix_dups():
        lane = jax.lax.broadcasted_iota(jnp.int32, (tn2, n1), 1)
        thr2 = jnp.where(ms[1] == ms[0], i1, -1)
        g2 = jnp.min(jnp.where((d == ms[1]) & (lane > thr2), lane, n1),
                     axis=1, keepdims=True)
        thr3 = jnp.where(ms[2] == ms[1], g2, -1)
        g3 = jnp.min(jnp.where((d == ms[2]) & (lane > thr3), lane, n1),
                     axis=1, keepdims=True)
        idx_ref[0] = (jnp.concatenate([i1, g2, g3], axis=1) + base).T

    w1 = 1.0 / (jnp.maximum(ms[0], 0.0) + _EPS)
    w2 = 1.0 / (jnp.maximum(ms[1], 0.0) + _EPS)
    w3 = 1.0 / (jnp.maximum(ms[2], 0.0) + _EPS)
    norm = w1 + w2 + w3
    wexp_ref[0] = jnp.concatenate(
        [jnp.broadcast_to(w1 / norm, (tn2, 16)),
         jnp.broadcast_to(w2 / norm, (tn2, 16)),
         jnp.broadcast_to(w3 / norm, (tn2, 16))], axis=1)


def _skip_add_body(z2_ref, ab2_ref, up_ref, y_ref):
    a2 = ab2_ref[:, 0:1]
    b2 = ab2_ref[:, 1:2]
    f2 = jnp.maximum(z2_ref[0] * a2 + b2, 0.0)
    y_ref[0] = f2 + up_ref[0].T


def _sc_interp(table, idx, wexp, qtot, tn2, cout):
    nblk = idx.shape[0]
    info = plsc.get_sparse_core_info()
    nw = info.num_cores * info.num_subcores
    blk_per_w = nblk // nw
    nh = tn2 // 128
    mesh = plsc.VectorSubcoreMesh(core_axis_name="c", subcore_axis_name="s")
    iw = 3 * tn2
    ww = tn2 * 48
    ow = tn2 * cout

    @functools.partial(
        pl.kernel, mesh=mesh,
        out_type=jax.ShapeDtypeStruct((qtot * cout,), jnp.float32),
        scratch_types=[
            pltpu.VMEM((iw,), jnp.int32),
            pltpu.VMEM((ww,), jnp.float32),
            pltpu.VMEM((128, cout), jnp.float32),
            pltpu.VMEM((ow,), jnp.float32),
            pltpu.SemaphoreType.DMA,
        ],
    )
    def k(table_hbm, idx_hbm, wexp_hbm, out_hbm, idx_v, w_v, rows_v, acc_v, sem):
        wid = lax.axis_index("s") * info.num_cores + lax.axis_index("c")
        ci = lax.iota(jnp.int32, 16)
        for blk in range(blk_per_w):
            bt = wid * blk_per_w + blk
            pltpu.sync_copy(idx_hbm.at[pl.ds(bt * iw, iw)], idx_v)
            pltpu.sync_copy(wexp_hbm.at[pl.ds(bt * ww, ww)], w_v)
            for kk in range(3):
                for h in range(nh):
                    pltpu.async_copy(
                        table_hbm.at[idx_v.at[pl.ds(kk * tn2 + h * 128, 128)]],
                        rows_v, sem).wait()

                    def qbody(q, _):
                        wv = w_v[pl.ds((h * 128 + q) * 48 + kk * 16, 16)]
                        for g in range(cout // 16):
                            seg = rows_v[q, pl.ds(g * 16, 16)] * wv
                            o = (h * 128 + q) * cout + g * 16
                            if kk == 0:
                                acc_v[pl.ds(o, 16)] = seg
                            else:
                                acc_v[pl.ds(o, 16)] += seg
                        return 0

                    lax.fori_loop(0, 128, qbody, 0)
            pltpu.sync_copy(acc_v, out_hbm.at[pl.ds(bt * ow, ow)])

    return k(table, idx.reshape(-1), wexp.reshape(-1))


def kernel(p1, x1, p2, x2, W1, gamma1, beta1, W2, gamma2, beta2):
    B, N1, _ = p1.shape
    _, Cin, _ = x1.shape
    _, Cskip, N2 = x2.shape
    Cout = W1.shape[0]
    TN2 = 256
    nt2 = N2 // TN2

    z1, z2, s1, q1, s2, q2 = pl.pallas_call(
        _feat_body,
        grid=(B,),
        in_specs=[
            pl.BlockSpec((1, Cin, N1), lambda b: (b, 0, 0)),
            pl.BlockSpec((Cout, Cin), lambda b: (0, 0)),
            pl.BlockSpec((1, Cskip, N2), lambda b: (b, 0, 0)),
            pl.BlockSpec((Cout, Cskip), lambda b: (0, 0)),
        ],
        out_specs=[
            pl.BlockSpec((1, N1, Cout), lambda b: (b, 0, 0)),
            pl.BlockSpec((1, Cout, N2), lambda b: (b, 0, 0)),
            pl.BlockSpec((1, Cout), lambda b: (0, 0)),
            pl.BlockSpec((1, Cout), lambda b: (0, 0)),
            pl.BlockSpec((Cout, 1), lambda b: (0, 0)),
            pl.BlockSpec((Cout, 1), lambda b: (0, 0)),
        ],
        out_shape=[
            jax.ShapeDtypeStruct((B, N1, Cout), jnp.float32),
            jax.ShapeDtypeStruct((B, Cout, N2), jnp.float32),
            jax.ShapeDtypeStruct((1, Cout), jnp.float32),
            jax.ShapeDtypeStruct((1, Cout), jnp.float32),
            jax.ShapeDtypeStruct((Cout, 1), jnp.float32),
            jax.ShapeDtypeStruct((Cout, 1), jnp.float32),
        ],
    )(x1, W1, x2, W2)

    cnt1 = jnp.float32(B * N1)
    mean1 = s1 / cnt1
    var1 = q1 / cnt1 - mean1 * mean1
    a1 = gamma1[None, :] / jnp.sqrt(var1 + 1e-5)
    b1 = beta1[None, :] - mean1 * a1
    ab1 = jnp.concatenate([a1, b1], axis=0)

    cnt2 = jnp.float32(B * N2)
    mean2 = s2 / cnt2
    var2 = q2 / cnt2 - mean2 * mean2
    a2 = gamma2[:, None] / jnp.sqrt(var2 + 1e-5)
    b2 = beta2[:, None] - mean2 * a2
    ab2 = jnp.concatenate([a2, b2], axis=1)

    idx, wexp, f1n = pl.pallas_call(
        functools.partial(_knn_idx_body, N1, TN2, nt2),
        grid=(B, nt2),
        in_specs=[
            pl.BlockSpec((1, N1, 3), lambda b, t: (b, 0, 0)),
            pl.BlockSpec((1, TN2, 3), lambda b, t: (b, t, 0)),
            pl.BlockSpec((1, N1, Cout), lambda b, t: (b, 0, 0)),
            pl.BlockSpec((2, Cout), lambda b, t: (0, 0)),
        ],
        out_specs=[
            pl.BlockSpec((1, 3, TN2), lambda b, t: (b * (N2 // TN2) + t, 0, 0)),
            pl.BlockSpec((1, TN2, 48), lambda b, t: (b * (N2 // TN2) + t, 0, 0)),
            pl.BlockSpec((1, N1, Cout), lambda b, t: (b, 0, 0)),
        ],
        out_shape=[
            jax.ShapeDtypeStruct((B * nt2, 3, TN2), jnp.int32),
            jax.ShapeDtypeStruct((B * nt2, TN2, 48), jnp.float32),
            jax.ShapeDtypeStruct((B, N1, Cout), jnp.float32),
        ],
        scratch_shapes=[pltpu.VMEM((N1, Cout), jnp.float32),
                        pltpu.VMEM((1, N1), jnp.float32)],
    )(p1, p2, z1, ab1)

    up = _sc_interp(f1n.reshape(B * N1, Cout), idx, wexp,
                    B * N2, TN2, Cout)

    y = pl.pallas_call(
        _skip_add_body,
        grid=(B, nt2),
        in_specs=[
            pl.BlockSpec((1, Cout, TN2), lambda b, t: (b, 0, t)),
            pl.BlockSpec((Cout, 2), lambda b, t: (0, 0)),
            pl.BlockSpec((1, TN2, Cout), lambda b, t: (b * (N2 // TN2) + t, 0, 0)),
        ],
        out_specs=pl.BlockSpec((1, Cout, TN2), lambda b, t: (b, 0, t)),
        out_shape=jax.ShapeDtypeStruct((B, Cout, N2), jnp.float32),
    )(z2, ab2, up.reshape(B * nt2, TN2, Cout))

    return (p2, y)


# speedup vs baseline: 1.1048x; 1.0654x over previous
import functools

import jax
import jax.numpy as jnp
from jax import lax
from jax.experimental import pallas as pl
from jax.experimental.pallas import tpu as pltpu
from jax.experimental.pallas import tpu_sc as plsc

_EPS = 1e-8
_BIG = 3.4e38


def _feat_body(x1_ref, w1_ref, x2_ref, w2_ref,
               z1_ref, z2_ref, s1_ref, q1_ref, s2_ref, q2_ref):
    b = pl.program_id(0)
    z1 = jax.lax.dot_general(x1_ref[0], w1_ref[...], (((0,), (1,)), ((), ())),
                             preferred_element_type=jnp.float32)
    z1_ref[0] = z1
    z2 = jax.lax.dot_general(w2_ref[...], x2_ref[0], (((1,), (0,)), ((), ())),
                             preferred_element_type=jnp.float32)
    z2_ref[0] = z2
    s1 = jnp.sum(z1, axis=0, keepdims=True)
    q1 = jnp.sum(z1 * z1, axis=0, keepdims=True)
    s2 = jnp.sum(z2, axis=1, keepdims=True)
    q2 = jnp.sum(z2 * z2, axis=1, keepdims=True)

    @pl.when(b == 0)
    def _init():
        s1_ref[...] = s1
        q1_ref[...] = q1
        s2_ref[...] = s2
        q2_ref[...] = q2

    @pl.when(b != 0)
    def _acc():
        s1_ref[...] += s1
        q1_ref[...] += q1
        s2_ref[...] += s2
        q2_ref[...] += q2


def _knn_idx_body(n1, tn2, nt2, p1_ref, p2_ref, z1_ref, ab1_ref,
                  idx_ref, wexp_ref, f1n_ref, f1n_scr, p1sq_ref):
    @pl.when(pl.program_id(0) == 0)
    def _prep():
        a1 = ab1_ref[0:1, :]
        b1 = ab1_ref[1:2, :]
        f1n_scr[...] = jnp.maximum(z1_ref[0] * a1 + b1, 0.0)
        f1n_ref[0] = f1n_scr[...]
        p1 = p1_ref[0]
        ones = jnp.ones((1, 3), jnp.float32)
        p1sq_ref[...] = jax.lax.dot_general(
            ones, p1 * p1, (((1,), (1,)), ((), ())),
            precision=jax.lax.Precision.HIGHEST,
            preferred_element_type=jnp.float32)

    p2t = p2_ref[0]
    p2sq = jnp.sum(p2t * p2t, axis=1, keepdims=True)
    dd = jax.lax.dot_general(p2t, p1_ref[0], (((1,), (1,)), ((), ())),
                             preferred_element_type=jnp.float32)
    d = p2sq + p1sq_ref[...] - 2.0 * dd

    big = jnp.full((tn2, 128), _BIG, jnp.float32)
    r1, r2, r3 = big, big, big
    for c in range(n1 // 128):
        x = d[:, c * 128:(c + 1) * 128]
        hi1 = jnp.maximum(r1, x)
        r1 = jnp.minimum(r1, x)
        hi2 = jnp.maximum(r2, hi1)
        r2 = jnp.minimum(r2, hi1)
        r3 = jnp.minimum(r3, hi2)
    r = jnp.concatenate([r1, r2, r3], axis=1)
    io = jax.lax.broadcasted_iota(jnp.int32, (tn2, 384), 1)
    ms = []
    for _ in range(3):
        m = jnp.min(r, axis=1, keepdims=True)
        i = jnp.min(jnp.where(r == m, io, 384), axis=1, keepdims=True)
        ms.append(m)
        r = jnp.where(io == i, _BIG, r)

    lane128 = jax.lax.broadcasted_iota(jnp.int32, (tn2, 128), 1)
    nfull = jnp.full((tn2, 128), n1, jnp.int32)
    j1, j2, j3 = nfull, nfull, nfull
    for c in range(n1 // 128):
        x = d[:, c * 128:(c + 1) * 128]
        lc = lane128 + (c * 128)
        j1 = jnp.minimum(j1, jnp.where(x == ms[0], lc, n1))
        j2 = jnp.minimum(j2, jnp.where(x == ms[1], lc, n1))
        j3 = jnp.minimum(j3, jnp.where(x == ms[2], lc, n1))
    i1 = jnp.min(j1, axis=1, keepdims=True)
    i2 = jnp.min(j2, axis=1, keepdims=True)
    i3 = jnp.min(j3, axis=1, keepdims=True)

    gidx = jnp.concatenate([i1, i2, i3], axis=1)
    idx_ref[0] = gidx.T

    dup = jnp.any((ms[1] == ms[0]) | (ms[2] == ms[1]))

    @pl.when(dup)
    def _fix_dups():
        lane = jax.lax.broadcasted_iota(jnp.int32, (tn2, n1), 1)
        thr2 = jnp.where(ms[1] == ms[0], i1, -1)
        g2 = jnp.min(jnp.where((d == ms[1]) & (lane > thr2), lane, n1),
                     axis=1, keepdims=True)
        thr3 = jnp.where(ms[2] == ms[1], g2, -1)
        g3 = jnp.min(jnp.where((d == ms[2]) & (lane > thr3), lane, n1),
                     axis=1, keepdims=True)
        idx_ref[0] = jnp.concatenate([i1, g2, g3], axis=1).T

    w1 = 1.0 / (jnp.maximum(ms[0], 0.0) + _EPS)
    w2 = 1.0 / (jnp.maximum(ms[1], 0.0) + _EPS)
    w3 = 1.0 / (jnp.maximum(ms[2], 0.0) + _EPS)
    norm = w1 + w2 + w3
    wexp_ref[0] = jnp.concatenate(
        [jnp.broadcast_to(w1 / norm, (tn2, 16)),
         jnp.broadcast_to(w2 / norm, (tn2, 16)),
         jnp.broadcast_to(w3 / norm, (tn2, 16))], axis=1)


def _skip_add_body(z2_ref, ab2_ref, up_ref, y_ref):
    a2 = ab2_ref[:, 0:1]
    b2 = ab2_ref[:, 1:2]
    f2 = jnp.maximum(z2_ref[0] * a2 + b2, 0.0)
    y_ref[0] = f2 + up_ref[0].T


def _sc_interp(table, idx, wexp, qtot, tn2, cout):
    nblk = idx.shape[0]
    info = plsc.get_sparse_core_info()
    nw = info.num_cores * info.num_subcores
    blk_per_w = nblk // nw
    nh = tn2 // 128
    mesh = plsc.VectorSubcoreMesh(core_axis_name="c", subcore_axis_name="s")
    iw = 3 * tn2
    ww = tn2 * 48
    ow = tn2 * cout

    @functools.partial(
        pl.kernel, mesh=mesh,
        out_type=jax.ShapeDtypeStruct((qtot * cout,), jnp.float32),
        scratch_types=[
            pltpu.VMEM((iw,), jnp.int32),
            pltpu.VMEM((ww,), jnp.float32),
            pltpu.VMEM((128, cout), jnp.float32),
            pltpu.VMEM((ow,), jnp.float32),
            pltpu.SemaphoreType.DMA,
        ],
    )
    def k(table_hbm, idx_hbm, wexp_hbm, out_hbm, idx_v, w_v, rows_v, acc_v, sem):
        wid = lax.axis_index("s") * info.num_cores + lax.axis_index("c")
        ci = lax.iota(jnp.int32, 16)
        for blk in range(blk_per_w):
            bt = wid * blk_per_w + blk
            pltpu.sync_copy(idx_hbm.at[pl.ds(bt * iw, iw)], idx_v)
            pltpu.sync_copy(wexp_hbm.at[pl.ds(bt * ww, ww)], w_v)
            for kk in range(3):
                for h in range(nh):
                    pltpu.async_copy(
                        table_hbm.at[idx_v.at[pl.ds(kk * tn2 + h * 128, 128)]],
                        rows_v, sem).wait()

                    def qbody(q, _):
                        wv = w_v[pl.ds((h * 128 + q) * 48 + kk * 16, 16)]
                        for g in range(cout // 16):
                            seg = rows_v[q, pl.ds(g * 16, 16)] * wv
                            o = (h * 128 + q) * cout + g * 16
                            if kk == 0:
                                acc_v[pl.ds(o, 16)] = seg
                            else:
                                acc_v[pl.ds(o, 16)] += seg
                        return 0

                    lax.fori_loop(0, 128, qbody, 0)
            pltpu.sync_copy(acc_v, out_hbm.at[pl.ds(bt * ow, ow)])

    return k(table, idx.reshape(-1), wexp.reshape(-1))


def kernel(p1, x1, p2, x2, W1, gamma1, beta1, W2, gamma2, beta2):
    B, N1, _ = p1.shape
    _, Cin, _ = x1.shape
    _, Cskip, N2 = x2.shape
    Cout = W1.shape[0]
    TN2 = 256
    nt2 = N2 // TN2

    z1, z2, s1, q1, s2, q2 = pl.pallas_call(
        _feat_body,
        grid=(B,),
        in_specs=[
            pl.BlockSpec((1, Cin, N1), lambda b: (b, 0, 0)),
            pl.BlockSpec((Cout, Cin), lambda b: (0, 0)),
            pl.BlockSpec((1, Cskip, N2), lambda b: (b, 0, 0)),
            pl.BlockSpec((Cout, Cskip), lambda b: (0, 0)),
        ],
        out_specs=[
            pl.BlockSpec((1, N1, Cout), lambda b: (b, 0, 0)),
            pl.BlockSpec((1, Cout, N2), lambda b: (b, 0, 0)),
            pl.BlockSpec((1, Cout), lambda b: (0, 0)),
            pl.BlockSpec((1, Cout), lambda b: (0, 0)),
            pl.BlockSpec((Cout, 1), lambda b: (0, 0)),
            pl.BlockSpec((Cout, 1), lambda b: (0, 0)),
        ],
        out_shape=[
            jax.ShapeDtypeStruct((B, N1, Cout), jnp.float32),
            jax.ShapeDtypeStruct((B, Cout, N2), jnp.float32),
            jax.ShapeDtypeStruct((1, Cout), jnp.float32),
            jax.ShapeDtypeStruct((1, Cout), jnp.float32),
            jax.ShapeDtypeStruct((Cout, 1), jnp.float32),
            jax.ShapeDtypeStruct((Cout, 1), jnp.float32),
        ],
    )(x1, W1, x2, W2)

    cnt1 = jnp.float32(B * N1)
    mean1 = s1 / cnt1
    var1 = q1 / cnt1 - mean1 * mean1
    a1 = gamma1[None, :] / jnp.sqrt(var1 + 1e-5)
    b1 = beta1[None, :] - mean1 * a1
    ab1 = jnp.concatenate([a1, b1], axis=0)

    cnt2 = jnp.float32(B * N2)
    mean2 = s2 / cnt2
    var2 = q2 / cnt2 - mean2 * mean2
    a2 = gamma2[:, None] / jnp.sqrt(var2 + 1e-5)
    b2 = beta2[:, None] - mean2 * a2
    ab2 = jnp.concatenate([a2, b2], axis=1)

    ys = []
    for bb in range(B):
        idx, wexp, f1n = pl.pallas_call(
            functools.partial(_knn_idx_body, N1, TN2, nt2),
            grid=(nt2,),
            in_specs=[
                pl.BlockSpec((1, N1, 3), lambda t, bb=bb: (bb, 0, 0)),
                pl.BlockSpec((1, TN2, 3), lambda t, bb=bb: (bb, t, 0)),
                pl.BlockSpec((1, N1, Cout), lambda t, bb=bb: (bb, 0, 0)),
                pl.BlockSpec((2, Cout), lambda t: (0, 0)),
            ],
            out_specs=[
                pl.BlockSpec((1, 3, TN2), lambda t: (t, 0, 0)),
                pl.BlockSpec((1, TN2, 48), lambda t: (t, 0, 0)),
                pl.BlockSpec((1, N1, Cout), lambda t: (0, 0, 0)),
            ],
            out_shape=[
                jax.ShapeDtypeStruct((nt2, 3, TN2), jnp.int32),
                jax.ShapeDtypeStruct((nt2, TN2, 48), jnp.float32),
                jax.ShapeDtypeStruct((1, N1, Cout), jnp.float32),
            ],
            scratch_shapes=[pltpu.VMEM((N1, Cout), jnp.float32),
                            pltpu.VMEM((1, N1), jnp.float32)],
        )(p1, p2, z1, ab1)

        up = _sc_interp(f1n.reshape(N1, Cout), idx, wexp,
                        N2, TN2, Cout)

        yb = pl.pallas_call(
            _skip_add_body,
            grid=(nt2,),
            in_specs=[
                pl.BlockSpec((1, Cout, TN2), lambda t, bb=bb: (bb, 0, t)),
                pl.BlockSpec((Cout, 2), lambda t: (0, 0)),
                pl.BlockSpec((1, TN2, Cout), lambda t: (t, 0, 0)),
            ],
            out_specs=pl.BlockSpec((1, Cout, TN2), lambda t: (0, 0, t)),
            out_shape=jax.ShapeDtypeStruct((1, Cout, N2), jnp.float32),
        )(z2, ab2, up.reshape(nt2, TN2, Cout))
        ys.append(yb)

    y = jnp.concatenate(ys, axis=0)
    return (p2, y)
